# baseline (speedup 1.0000x reference)
import functools

import jax
import jax.numpy as jnp
from jax import lax
from jax.experimental import pallas as pl
from jax.experimental.pallas import tpu as pltpu
from jax.experimental.pallas import tpu_sc as plsc

N_NODES = 10000
N_EDGES = 320000
D = 128
HID = 256
OUT = 128
NUM_LAYERS = 3
NUM_GRAPHS = 64

NC = 2
NS = 16
CHUNK = 128
NPAD = 10240
K0 = 160
TOTAL_CHUNKS = NS * K0
EDGES_PAD = TOTAL_CHUNKS * CHUNK
ROWS_PER_TILE = NPAD // NS
NBUF = 2
PHASE = 16

BM = 2000
GRID = N_NODES // BM


def _sc_scatter(m, src2d, dst2d):
    mesh = plsc.VectorSubcoreMesh(core_axis_name="c", subcore_axis_name="s")

    @functools.partial(
        pl.kernel,
        out_type=jax.ShapeDtypeStruct((NPAD, D), jnp.float32),
        mesh=mesh,
        scratch_types=[
            pltpu.VMEM((PHASE, CHUNK), jnp.int32),
            pltpu.VMEM((PHASE, CHUNK), jnp.int32),
            [pltpu.VMEM((CHUNK, D), jnp.float32)] * NBUF,
            pltpu.VMEM_SHARED((NPAD, D), jnp.float32),
            [pltpu.SemaphoreType.DMA] * NBUF,
        ],
    )
    def k(m_hbm, src_hbm, dst_hbm, out_hbm, src_v, dst_v, bufs,
          agg_sh, sems):
        c = lax.axis_index("c")
        s = lax.axis_index("s")

        @pl.when(c == 0)
        def _():
            zv = jnp.zeros((16,), jnp.float32)

            def zfill(t, carry):
                bufs[0][t // (D // 16), pl.ds((t % (D // 16)) * 16, 16)] = zv
                return carry
            lax.fori_loop(0, CHUNK * D // 16, zfill, 0)

            def zero_body(i, carry):
                pltpu.sync_copy(
                    bufs[0],
                    agg_sh.at[pl.ds(s * ROWS_PER_TILE + i * CHUNK, CHUNK)])
                return carry
            lax.fori_loop(0, ROWS_PER_TILE // CHUNK, zero_body, 0)
            plsc.subcore_barrier()

            def phase_body(phase, pcarry):
                base = s * K0 + phase * PHASE
                pltpu.sync_copy(src_hbm.at[pl.ds(base, PHASE)], src_v)
                pltpu.sync_copy(dst_hbm.at[pl.ds(base, PHASE)], dst_v)
                for b in range(NBUF):
                    pltpu.async_copy(m_hbm.at[src_v.at[b]], bufs[b],
                                     sems[b])

                def body(j, carry):
                    for b in range(NBUF):
                        i = j * NBUF + b
                        pltpu.make_async_copy(
                            m_hbm.at[src_v.at[i]], bufs[b], sems[b]).wait()
                        pltpu.sync_copy(bufs[b], agg_sh.at[dst_v.at[i]],
                                        add=True)

                        @pl.when(i + NBUF < PHASE)
                        def _():
                            pltpu.async_copy(
                                m_hbm.at[src_v.at[i + NBUF]], bufs[b],
                                sems[b])
                    return carry
                lax.fori_loop(0, PHASE // NBUF, body, 0)
                return pcarry
            lax.fori_loop(0, K0 // PHASE, phase_body, 0)

            plsc.subcore_barrier()
            pltpu.sync_copy(
                agg_sh.at[pl.ds(s * ROWS_PER_TILE, ROWS_PER_TILE)],
                out_hbm.at[pl.ds(s * ROWS_PER_TILE, ROWS_PER_TILE)])

    return k(m, src2d, dst2d)


def _pre_body(x_ref, w0_ref, m_ref):
    m_ref[...] = jnp.dot(x_ref[...], w0_ref[...],
                         preferred_element_type=jnp.float32)


def _gru(p, h, wih, bih, whh, bhh):
    gi = jnp.dot(p, wih, preferred_element_type=jnp.float32) + bih
    gh = jnp.dot(h, whh, preferred_element_type=jnp.float32) + bhh
    r = jax.nn.sigmoid(gi[:, :D] + gh[:, :D])
    z = jax.nn.sigmoid(gi[:, D:2 * D] + gh[:, D:2 * D])
    n = jnp.tanh(gi[:, 2 * D:] + r * gh[:, 2 * D:])
    return (1.0 - z) * n + z * h


def _layer_body(p_ref, h_ref, wih_ref, bih_ref, whh_ref, bhh_ref, wn_ref,
                hn_ref, m_ref):
    hn = _gru(p_ref[...], h_ref[...], wih_ref[...], bih_ref[...],
              whh_ref[...], bhh_ref[...])
    hn_ref[...] = hn
    m_ref[...] = jnp.dot(hn, wn_ref[...], preferred_element_type=jnp.float32)


def _final_body(p_ref, h_ref, wih_ref, bih_ref, whh_ref, bhh_ref, batch_ref,
                w1_ref, b1_ref, w2_ref, b2_ref, out_ref, pooled_ref):
    i = pl.program_id(0)

    @pl.when(i == 0)
    def _():
        pooled_ref[...] = jnp.zeros_like(pooled_ref)

    hn = _gru(p_ref[...], h_ref[...], wih_ref[...], bih_ref[...],
              whh_ref[...], bhh_ref[...])
    seg = batch_ref[...].reshape(1, BM)
    gids = lax.broadcasted_iota(jnp.int32, (NUM_GRAPHS, BM), 0)
    onehot = (gids == seg).astype(jnp.float32)
    pooled_ref[...] += jnp.dot(onehot, hn,
                               preferred_element_type=jnp.float32)

    @pl.when(i == GRID - 1)
    def _():
        hm = jax.nn.relu(
            jnp.dot(pooled_ref[...], w1_ref[...],
                    preferred_element_type=jnp.float32) + b1_ref[...])
        out_ref[...] = (
            jnp.dot(hm, w2_ref[...], preferred_element_type=jnp.float32)
            + b2_ref[...])


def _row_spec(width):
    return pl.BlockSpec((BM, width), lambda i: (i, 0))


def _full_spec(rows, cols):
    return pl.BlockSpec((rows, cols), lambda i: (0, 0))


def _pre_call(x, w0):
    return pl.pallas_call(
        _pre_body,
        grid=(GRID,),
        in_specs=[_row_spec(D), _full_spec(D, D)],
        out_specs=_row_spec(D),
        out_shape=jax.ShapeDtypeStruct((N_NODES, D), jnp.float32),
    )(x, w0)


def _layer_call(p, h, wih_t, bih2, whh_t, bhh2, wn):
    return pl.pallas_call(
        _layer_body,
        grid=(GRID,),
        in_specs=[_row_spec(D), _row_spec(D),
                  _full_spec(D, 3 * D), _full_spec(1, 3 * D),
                  _full_spec(D, 3 * D), _full_spec(1, 3 * D),
                  _full_spec(D, D)],
        out_specs=[_row_spec(D), _row_spec(D)],
        out_shape=[jax.ShapeDtypeStruct((N_NODES, D), jnp.float32),
                   jax.ShapeDtypeStruct((N_NODES, D), jnp.float32)],
    )(p, h, wih_t, bih2, whh_t, bhh2, wn)


def _final_call(p, h, wih_t, bih2, whh_t, bhh2, batch3, w1, b12, w2, b22):
    return pl.pallas_call(
        _final_body,
        grid=(GRID,),
        in_specs=[_row_spec(D), _row_spec(D),
                  _full_spec(D, 3 * D), _full_spec(1, 3 * D),
                  _full_spec(D, 3 * D), _full_spec(1, 3 * D),
                  pl.BlockSpec((1, 1, BM), lambda i: (i, 0, 0)),
                  _full_spec(D, HID), _full_spec(1, HID),
                  _full_spec(HID, OUT), _full_spec(1, OUT)],
        out_specs=pl.BlockSpec((NUM_GRAPHS, OUT), lambda i: (0, 0)),
        out_shape=jax.ShapeDtypeStruct((NUM_GRAPHS, OUT), jnp.float32),
        scratch_shapes=[pltpu.VMEM((NUM_GRAPHS, D), jnp.float32)],
    )(p, h, wih_t, bih2, whh_t, bhh2, batch3, w1, b12, w2, b22)


def kernel(x, edge_index, batch, W, W_ih, W_hh, b_ih, b_hh, W1, b1, W2, b2):
    src = edge_index[0]
    dst = edge_index[1]
    pad = EDGES_PAD - N_EDGES
    src2d = jnp.concatenate(
        [src, jnp.zeros((pad,), jnp.int32)]).reshape(TOTAL_CHUNKS, CHUNK)
    dst2d = jnp.concatenate(
        [dst, jnp.full((pad,), N_NODES, jnp.int32)]).reshape(
            TOTAL_CHUNKS, CHUNK)
    batch3 = batch.reshape(GRID, 1, BM)

    wih_t = W_ih.T
    whh_t = W_hh.T
    bih2 = b_ih.reshape(1, 3 * D)
    bhh2 = b_hh.reshape(1, 3 * D)
    b12 = b1.reshape(1, HID)
    b22 = b2.reshape(1, OUT)

    h = x
    m = _pre_call(x, W[0])
    for i in range(NUM_LAYERS):
        p = _sc_scatter(m, src2d, dst2d)
        if i < NUM_LAYERS - 1:
            h, m = _layer_call(p, h, wih_t, bih2, whh_t, bhh2, W[i + 1])
        else:
            out = _final_call(p, h, wih_t, bih2, whh_t, bhh2, batch3,
                              W1, b12, W2, b22)
    return out


# speedup vs baseline: 1.0169x; 1.0169x over previous
import functools

import jax
import jax.numpy as jnp
from jax import lax
from jax.experimental import pallas as pl
from jax.experimental.pallas import tpu as pltpu
from jax.experimental.pallas import tpu_sc as plsc

N_NODES = 10000
N_EDGES = 320000
D = 128
HID = 256
OUT = 128
NUM_LAYERS = 3
NUM_GRAPHS = 64

NC = 2
NS = 16
CHUNK = 128
NPAD = 10240
K0 = 160
TOTAL_CHUNKS = NS * K0
EDGES_PAD = TOTAL_CHUNKS * CHUNK
ROWS_PER_TILE = NPAD // NS
NBUF = 2
PHASE = 16

BM = 2000
GRID = N_NODES // BM


def _sc_scatter(m, src2d, dst2d):
    mesh = plsc.VectorSubcoreMesh(core_axis_name="c", subcore_axis_name="s")

    @functools.partial(
        pl.kernel,
        out_type=jax.ShapeDtypeStruct((NPAD, D), jnp.float32),
        mesh=mesh,
        scratch_types=[
            [pltpu.VMEM((PHASE, CHUNK), jnp.int32)] * 2,
            [pltpu.VMEM((PHASE, CHUNK), jnp.int32)] * 2,
            [pltpu.VMEM((CHUNK, D), jnp.float32)] * NBUF,
            pltpu.VMEM_SHARED((NPAD, D), jnp.float32),
            [pltpu.SemaphoreType.DMA] * NBUF,
            [pltpu.SemaphoreType.DMA] * 2,
        ],
    )
    def k(m_hbm, src_hbm, dst_hbm, out_hbm, srcs, dsts, bufs,
          agg_sh, sems, isems):
        c = lax.axis_index("c")
        s = lax.axis_index("s")
        nph = K0 // PHASE

        @pl.when(c == 0)
        def _():
            pltpu.async_copy(src_hbm.at[pl.ds(s * K0, PHASE)], srcs[0],
                             isems[0])
            pltpu.async_copy(dst_hbm.at[pl.ds(s * K0, PHASE)], dsts[0],
                             isems[0])

            zv = jnp.zeros((16,), jnp.float32)

            def zfill(t, carry):
                bufs[0][t // (D // 16), pl.ds((t % (D // 16)) * 16, 16)] = zv
                return carry
            lax.fori_loop(0, CHUNK * D // 16, zfill, 0)

            def zero_body(i, carry):
                pltpu.sync_copy(
                    bufs[0],
                    agg_sh.at[pl.ds(s * ROWS_PER_TILE + i * CHUNK, CHUNK)])
                return carry
            lax.fori_loop(0, ROWS_PER_TILE // CHUNK, zero_body, 0)
            plsc.subcore_barrier()

            def pair_body(q, pcarry):
                for pb in range(2):
                    ph = q * 2 + pb
                    base = s * K0 + ph * PHASE
                    src_v = srcs[pb]
                    dst_v = dsts[pb]
                    pltpu.make_async_copy(
                        src_hbm.at[pl.ds(base, PHASE)], src_v,
                        isems[pb]).wait()
                    pltpu.make_async_copy(
                        dst_hbm.at[pl.ds(base, PHASE)], dst_v,
                        isems[pb]).wait()

                    @pl.when(ph + 1 < nph)
                    def _():
                        nxt = s * K0 + (ph + 1) * PHASE
                        pltpu.async_copy(
                            src_hbm.at[pl.ds(nxt, PHASE)], srcs[1 - pb],
                            isems[1 - pb])
                        pltpu.async_copy(
                            dst_hbm.at[pl.ds(nxt, PHASE)], dsts[1 - pb],
                            isems[1 - pb])

                    for b in range(NBUF):
                        pltpu.async_copy(m_hbm.at[src_v.at[b]], bufs[b],
                                         sems[b])

                    def body(j, carry):
                        for b in range(NBUF):
                            i = j * NBUF + b
                            pltpu.make_async_copy(
                                m_hbm.at[src_v.at[i]], bufs[b],
                                sems[b]).wait()
                            pltpu.sync_copy(bufs[b], agg_sh.at[dst_v.at[i]],
                                            add=True)

                            @pl.when(i + NBUF < PHASE)
                            def _():
                                pltpu.async_copy(
                                    m_hbm.at[src_v.at[i + NBUF]], bufs[b],
                                    sems[b])
                        return carry
                    lax.fori_loop(0, PHASE // NBUF, body, 0)
                return pcarry
            lax.fori_loop(0, nph // 2, pair_body, 0)

            plsc.subcore_barrier()
            pltpu.sync_copy(
                agg_sh.at[pl.ds(s * ROWS_PER_TILE, ROWS_PER_TILE)],
                out_hbm.at[pl.ds(s * ROWS_PER_TILE, ROWS_PER_TILE)])

    return k(m, src2d, dst2d)


def _pre_body(x_ref, w0_ref, m_ref):
    m_ref[...] = jnp.dot(x_ref[...], w0_ref[...],
                         preferred_element_type=jnp.float32)


def _gru(p, h, wih, bih, whh, bhh):
    gi = jnp.dot(p, wih, preferred_element_type=jnp.float32) + bih
    gh = jnp.dot(h, whh, preferred_element_type=jnp.float32) + bhh
    r = jax.nn.sigmoid(gi[:, :D] + gh[:, :D])
    z = jax.nn.sigmoid(gi[:, D:2 * D] + gh[:, D:2 * D])
    n = jnp.tanh(gi[:, 2 * D:] + r * gh[:, 2 * D:])
    return (1.0 - z) * n + z * h


def _layer_body(p_ref, h_ref, wih_ref, bih_ref, whh_ref, bhh_ref, wn_ref,
                hn_ref, m_ref):
    hn = _gru(p_ref[...], h_ref[...], wih_ref[...], bih_ref[...],
              whh_ref[...], bhh_ref[...])
    hn_ref[...] = hn
    m_ref[...] = jnp.dot(hn, wn_ref[...], preferred_element_type=jnp.float32)


def _final_body(p_ref, h_ref, wih_ref, bih_ref, whh_ref, bhh_ref, batch_ref,
                w1_ref, b1_ref, w2_ref, b2_ref, out_ref, pooled_ref):
    i = pl.program_id(0)

    @pl.when(i == 0)
    def _():
        pooled_ref[...] = jnp.zeros_like(pooled_ref)

    hn = _gru(p_ref[...], h_ref[...], wih_ref[...], bih_ref[...],
              whh_ref[...], bhh_ref[...])
    seg = batch_ref[...].reshape(1, BM)
    gids = lax.broadcasted_iota(jnp.int32, (NUM_GRAPHS, BM), 0)
    onehot = (gids == seg).astype(jnp.float32)
    pooled_ref[...] += jnp.dot(onehot, hn,
                               preferred_element_type=jnp.float32)

    @pl.when(i == GRID - 1)
    def _():
        hm = jax.nn.relu(
            jnp.dot(pooled_ref[...], w1_ref[...],
                    preferred_element_type=jnp.float32) + b1_ref[...])
        out_ref[...] = (
            jnp.dot(hm, w2_ref[...], preferred_element_type=jnp.float32)
            + b2_ref[...])


def _row_spec(width):
    return pl.BlockSpec((BM, width), lambda i: (i, 0))


def _full_spec(rows, cols):
    return pl.BlockSpec((rows, cols), lambda i: (0, 0))


def _pre_call(x, w0):
    return pl.pallas_call(
        _pre_body,
        grid=(GRID,),
        in_specs=[_row_spec(D), _full_spec(D, D)],
        out_specs=_row_spec(D),
        out_shape=jax.ShapeDtypeStruct((N_NODES, D), jnp.float32),
    )(x, w0)


def _layer_call(p, h, wih_t, bih2, whh_t, bhh2, wn):
    return pl.pallas_call(
        _layer_body,
        grid=(GRID,),
        in_specs=[_row_spec(D), _row_spec(D),
                  _full_spec(D, 3 * D), _full_spec(1, 3 * D),
                  _full_spec(D, 3 * D), _full_spec(1, 3 * D),
                  _full_spec(D, D)],
        out_specs=[_row_spec(D), _row_spec(D)],
        out_shape=[jax.ShapeDtypeStruct((N_NODES, D), jnp.float32),
                   jax.ShapeDtypeStruct((N_NODES, D), jnp.float32)],
    )(p, h, wih_t, bih2, whh_t, bhh2, wn)


def _final_call(p, h, wih_t, bih2, whh_t, bhh2, batch3, w1, b12, w2, b22):
    return pl.pallas_call(
        _final_body,
        grid=(GRID,),
        in_specs=[_row_spec(D), _row_spec(D),
                  _full_spec(D, 3 * D), _full_spec(1, 3 * D),
                  _full_spec(D, 3 * D), _full_spec(1, 3 * D),
                  pl.BlockSpec((1, 1, BM), lambda i: (i, 0, 0)),
                  _full_spec(D, HID), _full_spec(1, HID),
                  _full_spec(HID, OUT), _full_spec(1, OUT)],
        out_specs=pl.BlockSpec((NUM_GRAPHS, OUT), lambda i: (0, 0)),
        out_shape=jax.ShapeDtypeStruct((NUM_GRAPHS, OUT), jnp.float32),
        scratch_shapes=[pltpu.VMEM((NUM_GRAPHS, D), jnp.float32)],
    )(p, h, wih_t, bih2, whh_t, bhh2, batch3, w1, b12, w2, b22)


def kernel(x, edge_index, batch, W, W_ih, W_hh, b_ih, b_hh, W1, b1, W2, b2):
    src = edge_index[0]
    dst = edge_index[1]
    pad = EDGES_PAD - N_EDGES
    src2d = jnp.concatenate(
        [src, jnp.zeros((pad,), jnp.int32)]).reshape(TOTAL_CHUNKS, CHUNK)
    dst2d = jnp.concatenate(
        [dst, jnp.full((pad,), N_NODES, jnp.int32)]).reshape(
            TOTAL_CHUNKS, CHUNK)
    batch3 = batch.reshape(GRID, 1, BM)

    wih_t = W_ih.T
    whh_t = W_hh.T
    bih2 = b_ih.reshape(1, 3 * D)
    bhh2 = b_hh.reshape(1, 3 * D)
    b12 = b1.reshape(1, HID)
    b22 = b2.reshape(1, OUT)

    h = x
    m = _pre_call(x, W[0])
    for i in range(NUM_LAYERS):
        p = _sc_scatter(m, src2d, dst2d)
        if i < NUM_LAYERS - 1:
            h, m = _layer_call(p, h, wih_t, bih2, whh_t, bhh2, W[i + 1])
        else:
            out = _final_call(p, h, wih_t, bih2, whh_t, bhh2, batch3,
                              W1, b12, W2, b22)
    return out


# speedup vs baseline: 1.4873x; 1.4626x over previous
import functools

import jax
import jax.numpy as jnp
from jax import lax
from jax.experimental import pallas as pl
from jax.experimental.pallas import tpu as pltpu
from jax.experimental.pallas import tpu_sc as plsc

N_NODES = 10000
N_EDGES = 320000
D = 128
HID = 256
OUT = 128
NUM_LAYERS = 3
NUM_GRAPHS = 64

NC = 2
NS = 16
NW = NC * NS
CHUNK = 128
NPAD = 10240
K0 = 144
K1 = 16
TOTAL_CHUNKS = NS * (K0 + K1)
EDGES_PAD = TOTAL_CHUNKS * CHUNK
ROWS_PER_TILE = NPAD // NS
NBUF = 2
PHASE = 16

BM = 2000
GRID = N_NODES // BM


def _sc_scatter_partials(m, src2d, dst2d):
    mesh = plsc.VectorSubcoreMesh(core_axis_name="c", subcore_axis_name="s")

    @functools.partial(
        pl.kernel,
        out_type=jax.ShapeDtypeStruct((NC, NPAD, D), jnp.float32),
        mesh=mesh,
        scratch_types=[
            pltpu.VMEM((PHASE, CHUNK), jnp.int32),
            pltpu.VMEM((PHASE, CHUNK), jnp.int32),
            [pltpu.VMEM((CHUNK, D), jnp.float32)] * NBUF,
            pltpu.VMEM_SHARED((NPAD, D), jnp.float32),
            [pltpu.SemaphoreType.DMA] * NBUF,
        ],
    )
    def k(m_hbm, src_hbm, dst_hbm, out_hbm, src_v, dst_v, bufs,
          agg_sh, sems):
        c = lax.axis_index("c")
        s = lax.axis_index("s")
        my_chunks = jnp.where(c == 0, K0, K1)
        my_base = jnp.where(c == 0, s * K0, NS * K0 + s * K1)

        zv = jnp.zeros((16,), jnp.float32)

        def zfill(t, carry):
            bufs[0][t // (D // 16), pl.ds((t % (D // 16)) * 16, 16)] = zv
            return carry
        lax.fori_loop(0, CHUNK * D // 16, zfill, 0)

        def zero_body(i, carry):
            pltpu.sync_copy(
                bufs[0],
                agg_sh.at[pl.ds(s * ROWS_PER_TILE + i * CHUNK, CHUNK)])
            return carry
        lax.fori_loop(0, ROWS_PER_TILE // CHUNK, zero_body, 0)
        plsc.subcore_barrier()

        def phase_body(phase, pcarry):
            base = my_base + phase * PHASE
            pltpu.sync_copy(src_hbm.at[pl.ds(base, PHASE)], src_v)
            pltpu.sync_copy(dst_hbm.at[pl.ds(base, PHASE)], dst_v)
            for b in range(NBUF):
                pltpu.async_copy(m_hbm.at[src_v.at[b]], bufs[b], sems[b])

            def body(j, carry):
                for b in range(NBUF):
                    i = j * NBUF + b
                    pltpu.make_async_copy(
                        m_hbm.at[src_v.at[i]], bufs[b], sems[b]).wait()
                    pltpu.sync_copy(bufs[b], agg_sh.at[dst_v.at[i]],
                                    add=True)

                    @pl.when(i + NBUF < PHASE)
                    def _():
                        pltpu.async_copy(
                            m_hbm.at[src_v.at[i + NBUF]], bufs[b],
                            sems[b])
                return carry
            lax.fori_loop(0, PHASE // NBUF, body, 0)
            return pcarry
        lax.fori_loop(0, my_chunks // PHASE, phase_body, 0)

        plsc.subcore_barrier()
        pltpu.sync_copy(
            agg_sh.at[pl.ds(s * ROWS_PER_TILE, ROWS_PER_TILE)],
            out_hbm.at[c, pl.ds(s * ROWS_PER_TILE, ROWS_PER_TILE)])

    return k(m, src2d, dst2d)


def _pre_body(x_ref, w0_ref, whh_ref, bhh_ref, m_ref, gh_ref):
    xb = x_ref[...]
    m_ref[...] = jnp.dot(xb, w0_ref[...], preferred_element_type=jnp.float32)
    gh_ref[...] = (
        jnp.dot(xb, whh_ref[...], preferred_element_type=jnp.float32)
        + bhh_ref[...])


def _gru(p0, p1, h, gh, wih, bih):
    agg = p0 + p1
    gi = jnp.dot(agg, wih, preferred_element_type=jnp.float32) + bih
    r = jax.nn.sigmoid(gi[:, :D] + gh[:, :D])
    z = jax.nn.sigmoid(gi[:, D:2 * D] + gh[:, D:2 * D])
    n = jnp.tanh(gi[:, 2 * D:] + r * gh[:, 2 * D:])
    return (1.0 - z) * n + z * h


def _layer_body(p0_ref, p1_ref, h_ref, gh_ref, wih_ref, bih_ref, wn_ref,
                whh_ref, bhh_ref, hn_ref, m_ref, ghn_ref):
    hn = _gru(p0_ref[...], p1_ref[...], h_ref[...], gh_ref[...],
              wih_ref[...], bih_ref[...])
    hn_ref[...] = hn
    m_ref[...] = jnp.dot(hn, wn_ref[...], preferred_element_type=jnp.float32)
    ghn_ref[...] = (
        jnp.dot(hn, whh_ref[...], preferred_element_type=jnp.float32)
        + bhh_ref[...])


def _final_body(p0_ref, p1_ref, h_ref, gh_ref, wih_ref, bih_ref, batch_ref,
                w1_ref, b1_ref, w2_ref, b2_ref, out_ref, pooled_ref):
    i = pl.program_id(0)

    @pl.when(i == 0)
    def _():
        pooled_ref[...] = jnp.zeros_like(pooled_ref)

    hn = _gru(p0_ref[...], p1_ref[...], h_ref[...], gh_ref[...],
              wih_ref[...], bih_ref[...])
    seg = batch_ref[...].reshape(1, BM)
    gids = lax.broadcasted_iota(jnp.int32, (NUM_GRAPHS, BM), 0)
    onehot = (gids == seg).astype(jnp.float32)
    pooled_ref[...] += jnp.dot(onehot, hn,
                               preferred_element_type=jnp.float32)

    @pl.when(i == GRID - 1)
    def _():
        hm = jax.nn.relu(
            jnp.dot(pooled_ref[...], w1_ref[...],
                    preferred_element_type=jnp.float32) + b1_ref[...])
        out_ref[...] = (
            jnp.dot(hm, w2_ref[...], preferred_element_type=jnp.float32)
            + b2_ref[...])


def _row_spec(width):
    return pl.BlockSpec((BM, width), lambda i: (i, 0))


def _full_spec(rows, cols):
    return pl.BlockSpec((rows, cols), lambda i: (0, 0))


def _pre_call(x, w0, whh_t, bhh2):
    return pl.pallas_call(
        _pre_body,
        grid=(GRID,),
        in_specs=[_row_spec(D), _full_spec(D, D), _full_spec(D, 3 * D),
                  _full_spec(1, 3 * D)],
        out_specs=[_row_spec(D), _row_spec(3 * D)],
        out_shape=[jax.ShapeDtypeStruct((N_NODES, D), jnp.float32),
                   jax.ShapeDtypeStruct((N_NODES, 3 * D), jnp.float32)],
    )(x, w0, whh_t, bhh2)


def _layer_call(p0, p1, h, gh, wih_t, bih2, wn, whh_t, bhh2):
    return pl.pallas_call(
        _layer_body,
        grid=(GRID,),
        in_specs=[_row_spec(D), _row_spec(D), _row_spec(D), _row_spec(3 * D),
                  _full_spec(D, 3 * D), _full_spec(1, 3 * D),
                  _full_spec(D, D), _full_spec(D, 3 * D),
                  _full_spec(1, 3 * D)],
        out_specs=[_row_spec(D), _row_spec(D), _row_spec(3 * D)],
        out_shape=[jax.ShapeDtypeStruct((N_NODES, D), jnp.float32),
                   jax.ShapeDtypeStruct((N_NODES, D), jnp.float32),
                   jax.ShapeDtypeStruct((N_NODES, 3 * D), jnp.float32)],
    )(p0, p1, h, gh, wih_t, bih2, wn, whh_t, bhh2)


def _final_call(p0, p1, h, gh, wih_t, bih2, batch3, w1, b12, w2, b22):
    return pl.pallas_call(
        _final_body,
        grid=(GRID,),
        in_specs=[_row_spec(D), _row_spec(D), _row_spec(D), _row_spec(3 * D),
                  _full_spec(D, 3 * D), _full_spec(1, 3 * D),
                  pl.BlockSpec((1, 1, BM), lambda i: (i, 0, 0)),
                  _full_spec(D, HID), _full_spec(1, HID),
                  _full_spec(HID, OUT), _full_spec(1, OUT)],
        out_specs=pl.BlockSpec((NUM_GRAPHS, OUT), lambda i: (0, 0)),
        out_shape=jax.ShapeDtypeStruct((NUM_GRAPHS, OUT), jnp.float32),
        scratch_shapes=[pltpu.VMEM((NUM_GRAPHS, D), jnp.float32)],
    )(p0, p1, h, gh, wih_t, bih2, batch3, w1, b12, w2, b22)


def kernel(x, edge_index, batch, W, W_ih, W_hh, b_ih, b_hh, W1, b1, W2, b2):
    src = edge_index[0]
    dst = edge_index[1]
    pad = EDGES_PAD - N_EDGES
    src2d = jnp.concatenate(
        [src, jnp.zeros((pad,), jnp.int32)]).reshape(TOTAL_CHUNKS, CHUNK)
    dst2d = jnp.concatenate(
        [dst, jnp.full((pad,), N_NODES, jnp.int32)]).reshape(
            TOTAL_CHUNKS, CHUNK)
    batch3 = batch.reshape(GRID, 1, BM)

    wih_t = W_ih.T
    whh_t = W_hh.T
    bih2 = b_ih.reshape(1, 3 * D)
    bhh2 = b_hh.reshape(1, 3 * D)
    b12 = b1.reshape(1, HID)
    b22 = b2.reshape(1, OUT)

    h = x
    m, gh = _pre_call(x, W[0], whh_t, bhh2)
    for i in range(NUM_LAYERS):
        parts = _sc_scatter_partials(m, src2d, dst2d)
        p0 = parts[0]
        p1 = parts[1]
        if i < NUM_LAYERS - 1:
            h, m, gh = _layer_call(p0, p1, h, gh, wih_t, bih2, W[i + 1],
                                   whh_t, bhh2)
        else:
            out = _final_call(p0, p1, h, gh, wih_t, bih2, batch3,
                              W1, b12, W2, b22)
    return out
